# Initial kernel scaffold; baseline (speedup 1.0000x reference)
import jax
import jax.numpy as jnp
from jax.experimental import pallas as pl


def kernel(query_trace, memory_matrix, steps):
    raise NotImplementedError("write your pallas kernel here")


# speedup vs baseline: 1.2124x; 1.2124x over previous
import functools

import jax
import jax.numpy as jnp
from jax.experimental import pallas as pl
from jax.experimental.pallas import tpu as pltpu

_CAP = 50000
_DIM = 1024
_B = 8
_BM = 2000
_EPS = 1e-8


def _step_body(cur_ref, mem_ref, out_ref, m_ref, l_ref, acc_ref):
    j = pl.program_id(0)
    nb = pl.num_programs(0)

    @pl.when(j == 0)
    def _init():
        m_ref[:] = jnp.full_like(m_ref, -jnp.inf)
        l_ref[:] = jnp.zeros_like(l_ref)
        acc_ref[:] = jnp.zeros_like(acc_ref)

    cur = cur_ref[:]
    block = mem_ref[:]

    cn = jnp.sqrt(jnp.sum(cur * cur, axis=1, keepdims=True))
    ncur = cur / jnp.maximum(cn, _EPS)
    mn = jnp.sqrt(jnp.sum(block * block, axis=1, keepdims=True))
    nblock = block / jnp.maximum(mn, _EPS)

    sim = jax.lax.dot_general(
        ncur, nblock, (((1,), (1,)), ((), ())),
        preferred_element_type=jnp.float32)

    m_old = m_ref[:, :1]
    m_new = jnp.maximum(m_old, jnp.max(sim, axis=1, keepdims=True))
    p = jnp.exp(sim - m_new)
    corr = jnp.exp(m_old - m_new)
    l_new = l_ref[:, :1] * corr + jnp.sum(p, axis=1, keepdims=True)
    pv = jax.lax.dot_general(
        p, block, (((1,), (0,)), ((), ())),
        preferred_element_type=jnp.float32)
    acc_new = acc_ref[:] * corr + pv

    m_ref[:] = jnp.broadcast_to(m_new, m_ref.shape)
    l_ref[:] = jnp.broadcast_to(l_new, l_ref.shape)
    acc_ref[:] = acc_new

    @pl.when(j == nb - 1)
    def _finalize():
        attracted = acc_ref[:] / l_ref[:, :1]
        out_ref[:] = 0.8 * attracted + 0.2 * cur


@functools.partial(jax.jit, static_argnames=("bm",))
def _one_step(cur, memory_matrix, bm=_BM):
    nb = _CAP // bm
    return pl.pallas_call(
        _step_body,
        grid=(nb,),
        in_specs=[
            pl.BlockSpec((_B, _DIM), lambda j: (0, 0)),
            pl.BlockSpec((bm, _DIM), lambda j: (j, 0)),
        ],
        out_specs=pl.BlockSpec((_B, _DIM), lambda j: (0, 0)),
        out_shape=jax.ShapeDtypeStruct((_B, _DIM), jnp.float32),
        scratch_shapes=[
            pltpu.VMEM((_B, 128), jnp.float32),
            pltpu.VMEM((_B, 128), jnp.float32),
            pltpu.VMEM((_B, _DIM), jnp.float32),
        ],
        compiler_params=pltpu.CompilerParams(
            dimension_semantics=("arbitrary",),
        ),
    )(cur, memory_matrix)


def kernel(query_trace, memory_matrix, steps):
    def body(_, cur):
        return _one_step(cur, memory_matrix)
    return jax.lax.fori_loop(0, steps, body, query_trace)


# speedup vs baseline: 1.2570x; 1.0368x over previous
import jax
import jax.numpy as jnp
from jax.experimental import pallas as pl
from jax.experimental.pallas import tpu as pltpu

_CAP = 50000
_DIM = 1024
_B = 8
_BM = 2000
_INV_EPS = 1e8


def _online_softmax_update(cur, block, sim, j, nb, out_ref, m_ref, l_ref,
                           acc_ref):
    m_old = m_ref[:, :1]
    m_new = jnp.maximum(m_old, jnp.max(sim, axis=1, keepdims=True))
    p = jnp.exp(sim - m_new)
    corr = jnp.exp(m_old - m_new)
    l_new = l_ref[:, :1] * corr + jnp.sum(p, axis=1, keepdims=True)
    pv = jax.lax.dot_general(
        p, block, (((1,), (0,)), ((), ())),
        preferred_element_type=jnp.float32)
    acc_new = acc_ref[:] * corr + pv

    m_ref[:] = jnp.broadcast_to(m_new, m_ref.shape)
    l_ref[:] = jnp.broadcast_to(l_new, l_ref.shape)
    acc_ref[:] = acc_new

    @pl.when(j == nb - 1)
    def _finalize():
        attracted = acc_ref[:] / l_ref[:, :1]
        out_ref[:] = 0.8 * attracted + 0.2 * cur


def _init_state(j, m_ref, l_ref, acc_ref):
    @pl.when(j == 0)
    def _init():
        m_ref[:] = jnp.full_like(m_ref, -jnp.inf)
        l_ref[:] = jnp.zeros_like(l_ref)
        acc_ref[:] = jnp.zeros_like(acc_ref)


def _inv_cn(cur):
    cn2 = jnp.sum(cur * cur, axis=1, keepdims=True)
    return jnp.minimum(jax.lax.rsqrt(cn2), _INV_EPS)


def _step0_body(cur_ref, mem_ref, out_ref, invmn_ref, m_ref, l_ref, acc_ref):
    j = pl.program_id(0)
    nb = pl.num_programs(0)
    _init_state(j, m_ref, l_ref, acc_ref)

    cur = cur_ref[:]
    block = mem_ref[:]

    sq = block * block
    mn2 = jax.lax.dot_general(
        jnp.ones((1, _DIM), jnp.float32), sq, (((1,), (1,)), ((), ())),
        preferred_element_type=jnp.float32)
    inv_mn = jnp.minimum(jax.lax.rsqrt(mn2), _INV_EPS)
    invmn_ref[:] = inv_mn[None]

    dots = jax.lax.dot_general(
        cur, block, (((1,), (1,)), ((), ())),
        preferred_element_type=jnp.float32)
    sim = dots * _inv_cn(cur) * inv_mn
    _online_softmax_update(cur, block, sim, j, nb, out_ref, m_ref, l_ref,
                           acc_ref)


def _stepn_body(cur_ref, mem_ref, invmn_ref, out_ref, m_ref, l_ref, acc_ref):
    j = pl.program_id(0)
    nb = pl.num_programs(0)
    _init_state(j, m_ref, l_ref, acc_ref)

    cur = cur_ref[:]
    block = mem_ref[:]

    dots = jax.lax.dot_general(
        cur, block, (((1,), (1,)), ((), ())),
        preferred_element_type=jnp.float32)
    sim = dots * _inv_cn(cur) * invmn_ref[0]
    _online_softmax_update(cur, block, sim, j, nb, out_ref, m_ref, l_ref,
                           acc_ref)


_SCRATCH = [
    pltpu.VMEM((_B, 128), jnp.float32),
    pltpu.VMEM((_B, 128), jnp.float32),
    pltpu.VMEM((_B, _DIM), jnp.float32),
]


def _step0(cur, memory_matrix):
    nb = _CAP // _BM
    return pl.pallas_call(
        _step0_body,
        grid=(nb,),
        in_specs=[
            pl.BlockSpec((_B, _DIM), lambda j: (0, 0)),
            pl.BlockSpec((_BM, _DIM), lambda j: (j, 0)),
        ],
        out_specs=[
            pl.BlockSpec((_B, _DIM), lambda j: (0, 0)),
            pl.BlockSpec((1, 1, _BM), lambda j: (j, 0, 0)),
        ],
        out_shape=[
            jax.ShapeDtypeStruct((_B, _DIM), jnp.float32),
            jax.ShapeDtypeStruct((_CAP // _BM, 1, _BM), jnp.float32),
        ],
        scratch_shapes=_SCRATCH,
        compiler_params=pltpu.CompilerParams(
            dimension_semantics=("arbitrary",),
        ),
    )(cur, memory_matrix)


def _stepn(cur, memory_matrix, inv_mn):
    nb = _CAP // _BM
    return pl.pallas_call(
        _stepn_body,
        grid=(nb,),
        in_specs=[
            pl.BlockSpec((_B, _DIM), lambda j: (0, 0)),
            pl.BlockSpec((_BM, _DIM), lambda j: (j, 0)),
            pl.BlockSpec((1, 1, _BM), lambda j: (j, 0, 0)),
        ],
        out_specs=pl.BlockSpec((_B, _DIM), lambda j: (0, 0)),
        out_shape=jax.ShapeDtypeStruct((_B, _DIM), jnp.float32),
        scratch_shapes=_SCRATCH,
        compiler_params=pltpu.CompilerParams(
            dimension_semantics=("arbitrary",),
        ),
    )(cur, memory_matrix, inv_mn)


def kernel(query_trace, memory_matrix, steps):
    cur1, inv_mn = _step0(query_trace, memory_matrix)

    def body(_, cur):
        return _stepn(cur, memory_matrix, inv_mn)

    cur = jax.lax.fori_loop(1, steps, body, cur1)
    return jnp.where(steps >= 1, cur, query_trace)


# speedup vs baseline: 1.4124x; 1.1236x over previous
import jax
import jax.numpy as jnp
from jax.experimental import pallas as pl
from jax.experimental.pallas import tpu as pltpu

_CAP = 50000
_DIM = 1024
_B = 8
_BM = 2000
_INV_EPS = 1e8


def _online_softmax_update(cur, block, sim, j, nb, out_ref, m_ref, l_ref,
                           acc_ref):
    m_old = m_ref[:, :1]
    m_new = jnp.maximum(m_old, jnp.max(sim, axis=1, keepdims=True))
    p = jnp.exp(sim - m_new)
    corr = jnp.exp(m_old - m_new)
    l_new = l_ref[:, :1] * corr + jnp.sum(p, axis=1, keepdims=True)
    pv = jax.lax.dot_general(
        p, block, (((1,), (0,)), ((), ())),
        preferred_element_type=jnp.float32)
    acc_new = acc_ref[:] * corr + pv

    m_ref[:] = jnp.broadcast_to(m_new, m_ref.shape)
    l_ref[:] = jnp.broadcast_to(l_new, l_ref.shape)
    acc_ref[:] = acc_new

    @pl.when(j == nb - 1)
    def _finalize():
        attracted = acc_ref[:] / l_ref[:, :1]
        out_ref[:] = 0.8 * attracted + 0.2 * cur


def _init_state(j, m_ref, l_ref, acc_ref):
    @pl.when(j == 0)
    def _init():
        m_ref[:] = jnp.full_like(m_ref, -jnp.inf)
        l_ref[:] = jnp.zeros_like(l_ref)
        acc_ref[:] = jnp.zeros_like(acc_ref)


def _inv_cn(cur):
    cn2 = jnp.sum(cur * cur, axis=1, keepdims=True)
    return jnp.minimum(jax.lax.rsqrt(cn2), _INV_EPS)


def _step0_body(cur_ref, mem_ref, out_ref, invmn_ref, membf_ref, m_ref, l_ref,
                acc_ref):
    j = pl.program_id(0)
    nb = pl.num_programs(0)
    _init_state(j, m_ref, l_ref, acc_ref)

    cur = cur_ref[:]
    block = mem_ref[:]
    membf_ref[:] = block.astype(jnp.bfloat16)

    sq = block * block
    mn2 = jax.lax.dot_general(
        jnp.ones((1, _DIM), jnp.float32), sq, (((1,), (1,)), ((), ())),
        preferred_element_type=jnp.float32)
    inv_mn = jnp.minimum(jax.lax.rsqrt(mn2), _INV_EPS)
    invmn_ref[:] = inv_mn[None]

    dots = jax.lax.dot_general(
        cur, block, (((1,), (1,)), ((), ())),
        preferred_element_type=jnp.float32)
    sim = dots * _inv_cn(cur) * inv_mn
    _online_softmax_update(cur, block, sim, j, nb, out_ref, m_ref, l_ref,
                           acc_ref)


def _stepn_body(cur_ref, mem_ref, invmn_ref, out_ref, m_ref, l_ref, acc_ref):
    j = pl.program_id(0)
    nb = pl.num_programs(0)
    _init_state(j, m_ref, l_ref, acc_ref)

    cur = cur_ref[:]
    block = mem_ref[:]

    dots = jax.lax.dot_general(
        cur.astype(jnp.bfloat16), block, (((1,), (1,)), ((), ())),
        preferred_element_type=jnp.float32)
    sim = dots * _inv_cn(cur) * invmn_ref[0]
    m_old = m_ref[:, :1]
    m_new = jnp.maximum(m_old, jnp.max(sim, axis=1, keepdims=True))
    p = jnp.exp(sim - m_new)
    corr = jnp.exp(m_old - m_new)
    l_new = l_ref[:, :1] * corr + jnp.sum(p, axis=1, keepdims=True)
    pv = jax.lax.dot_general(
        p.astype(jnp.bfloat16), block, (((1,), (0,)), ((), ())),
        preferred_element_type=jnp.float32)
    acc_new = acc_ref[:] * corr + pv

    m_ref[:] = jnp.broadcast_to(m_new, m_ref.shape)
    l_ref[:] = jnp.broadcast_to(l_new, l_ref.shape)
    acc_ref[:] = acc_new

    @pl.when(j == nb - 1)
    def _finalize():
        attracted = acc_ref[:] / l_ref[:, :1]
        out_ref[:] = 0.8 * attracted + 0.2 * cur


_SCRATCH = [
    pltpu.VMEM((_B, 128), jnp.float32),
    pltpu.VMEM((_B, 128), jnp.float32),
    pltpu.VMEM((_B, _DIM), jnp.float32),
]


def _step0(cur, memory_matrix):
    nb = _CAP // _BM
    return pl.pallas_call(
        _step0_body,
        grid=(nb,),
        in_specs=[
            pl.BlockSpec((_B, _DIM), lambda j: (0, 0)),
            pl.BlockSpec((_BM, _DIM), lambda j: (j, 0)),
        ],
        out_specs=[
            pl.BlockSpec((_B, _DIM), lambda j: (0, 0)),
            pl.BlockSpec((1, 1, _BM), lambda j: (j, 0, 0)),
            pl.BlockSpec((_BM, _DIM), lambda j: (j, 0)),
        ],
        out_shape=[
            jax.ShapeDtypeStruct((_B, _DIM), jnp.float32),
            jax.ShapeDtypeStruct((_CAP // _BM, 1, _BM), jnp.float32),
            jax.ShapeDtypeStruct((_CAP, _DIM), jnp.bfloat16),
        ],
        scratch_shapes=_SCRATCH,
        compiler_params=pltpu.CompilerParams(
            dimension_semantics=("arbitrary",),
        ),
    )(cur, memory_matrix)


def _stepn(cur, memory_matrix, inv_mn):
    nb = _CAP // _BM
    return pl.pallas_call(
        _stepn_body,
        grid=(nb,),
        in_specs=[
            pl.BlockSpec((_B, _DIM), lambda j: (0, 0)),
            pl.BlockSpec((_BM, _DIM), lambda j: (j, 0)),
            pl.BlockSpec((1, 1, _BM), lambda j: (j, 0, 0)),
        ],
        out_specs=pl.BlockSpec((_B, _DIM), lambda j: (0, 0)),
        out_shape=jax.ShapeDtypeStruct((_B, _DIM), jnp.float32),
        scratch_shapes=_SCRATCH,
        compiler_params=pltpu.CompilerParams(
            dimension_semantics=("arbitrary",),
        ),
    )(cur, memory_matrix, inv_mn)


def kernel(query_trace, memory_matrix, steps):
    cur1, inv_mn, mem_bf16 = _step0(query_trace, memory_matrix)

    def body(_, cur):
        return _stepn(cur, mem_bf16, inv_mn)

    cur = jax.lax.fori_loop(1, steps, body, cur1)
    return jnp.where(steps >= 1, cur, query_trace)
